# initial kernel scaffold (unmeasured)
import jax
import jax.numpy as jnp
from jax import lax
from jax.experimental import pallas as pl
from jax.experimental.pallas import tpu as pltpu

N_DEV = 4
M_LOC = 1024
K = 4096
N_TOT = 8192
N_BLK = N_TOT // N_DEV
W_TILE = 256
N_TILES = N_TOT // W_TILE
TILES_PER_BLK = N_BLK // W_TILE

_PRECISION = lax.Precision.HIGHEST


def kernel(x, w_mat):
    def body(x_ref, w_ref, out_ref,
             w_buf, y_blk, q_send, q_recv, amax_mine, amax_all,
             y_hbm,
             w_sems, y_sems, r_sems, o_sems,
             ax_send_sems, ax_recv_sems, d_send_sems, d_recv_sems):
        my = lax.axis_index("i")

        bsem = pltpu.get_barrier_semaphore()
        for h in (1, 2, 3):
            pl.semaphore_signal(
                bsem, inc=1,
                device_id=((my + h) % N_DEV,),
                device_id_type=pl.DeviceIdType.MESH,
            )
        pl.semaphore_wait(bsem, 3)

        amax_all[...] = jnp.zeros((N_DEV, 8, 128), jnp.float32)

        def w_copy(t, slot):
            return pltpu.make_async_copy(
                w_ref.at[:, pl.ds(t * W_TILE, W_TILE)],
                w_buf.at[slot],
                w_sems.at[slot],
            )

        def y_store(j):
            return pltpu.make_async_copy(
                y_blk.at[j % 2], y_hbm.at[j], y_sems.at[j % 2])

        w_copy(0, 0).start()
        amax = jnp.float32(0.0)
        for t in range(N_TILES):
            slot = t % 2
            if t + 1 < N_TILES:
                w_copy(t + 1, (t + 1) % 2).start()
            w_copy(t, slot).wait()
            part = lax.dot_general(
                x_ref[...], w_buf[slot],
                dimension_numbers=(((1,), (0,)), ((), ())),
                precision=_PRECISION,
                preferred_element_type=jnp.float32,
            )
            j, tj = divmod(t, TILES_PER_BLK)
            if tj == 0 and j >= 2:
                y_store(j - 2).wait()
            y_blk[j % 2, :, tj * W_TILE:(tj + 1) * W_TILE] = part
            amax = jnp.maximum(amax, jnp.max(jnp.abs(part)))
            if tj == TILES_PER_BLK - 1:
                y_store(j).start()

        amax_mine[...] = jnp.full((8, 128), amax, jnp.float32)
        for h in (1, 2, 3):
            pltpu.make_async_remote_copy(
                src_ref=amax_mine,
                dst_ref=amax_all.at[N_DEV - h],
                send_sem=ax_send_sems.at[h],
                recv_sem=ax_recv_sems.at[N_DEV - h],
                device_id=((my + h) % N_DEV,),
                device_id_type=pl.DeviceIdType.MESH,
            ).start()
        for r in (1, 2, 3):
            pltpu.make_async_remote_copy(
                src_ref=amax_mine,
                dst_ref=amax_all.at[r],
                send_sem=ax_send_sems.at[0],
                recv_sem=ax_recv_sems.at[r],
                device_id=(my,),
                device_id_type=pl.DeviceIdType.MESH,
            ).wait_recv()
        amax_g = jnp.maximum(amax, jnp.max(amax_all[...]))
        inv = 448.0 / amax_g
        scale = amax_g / 448.0

        def quant(v):
            return jnp.clip(v * inv, -448.0, 448.0).astype(jnp.float8_e4m3fn)

        for j in (2, 3):
            y_store(j).wait()

        def y_read(jb, slot):
            return pltpu.make_async_copy(
                y_hbm.at[jb], y_blk.at[slot], r_sems.at[slot])

        y_read((my + 1) % N_DEV, 0).start()
        y_read((my + 2) % N_DEV, 1).start()
        for h in (1, 2, 3):
            slot = (h - 1) % 2
            y_read((my + h) % N_DEV, slot).wait()
            q_send[h - 1, :, :] = quant(y_blk[slot])
            pltpu.make_async_remote_copy(
                src_ref=q_send.at[h - 1],
                dst_ref=q_recv.at[N_DEV - h],
                send_sem=d_send_sems.at[h],
                recv_sem=d_recv_sems.at[N_DEV - h],
                device_id=((my + h) % N_DEV,),
                device_id_type=pl.DeviceIdType.MESH,
            ).start()
            if h == 1:
                y_read((my + 3) % N_DEV, 0).start()
            elif h == 2:
                y_read(my, 1).start()

        def out_store(s_idx, slot):
            return pltpu.make_async_copy(
                y_blk.at[slot],
                out_ref.at[pl.ds(s_idx * M_LOC, M_LOC), :],
                o_sems.at[slot],
            )

        y_read(my, 1).wait()
        y_blk[1, :, :] = quant(y_blk[1]).astype(jnp.float32) * scale
        out_store(my, 1).start()

        prev = [None, None, out_store(my, 1)]
        for n, r in enumerate((3, 2, 1)):
            slot = n % 2
            pltpu.make_async_remote_copy(
                src_ref=q_send.at[0],
                dst_ref=q_recv.at[r],
                send_sem=d_send_sems.at[0],
                recv_sem=d_recv_sems.at[r],
                device_id=(my,),
                device_id_type=pl.DeviceIdType.MESH,
            ).wait_recv()
            if prev[slot] is not None:
                prev[slot].wait()
            y_blk[slot, :, :] = q_recv[r].astype(jnp.float32) * scale
            st = out_store((my + r) % N_DEV, slot)
            st.start()
            prev[slot] = st
        prev[0].wait()
        prev[1].wait()
        prev[2].wait()

        for h in (1, 2, 3):
            pltpu.make_async_remote_copy(
                src_ref=amax_mine,
                dst_ref=amax_all.at[N_DEV - h],
                send_sem=ax_send_sems.at[h],
                recv_sem=ax_recv_sems.at[0],
                device_id=((my + h) % N_DEV,),
                device_id_type=pl.DeviceIdType.MESH,
            ).wait_send()
            pltpu.make_async_remote_copy(
                src_ref=q_send.at[h - 1],
                dst_ref=q_recv.at[N_DEV - h],
                send_sem=d_send_sems.at[h],
                recv_sem=d_recv_sems.at[0],
                device_id=((my + h) % N_DEV,),
                device_id_type=pl.DeviceIdType.MESH,
            ).wait_send()

    return pl.pallas_call(
        body,
        out_shape=jax.ShapeDtypeStruct((N_DEV * M_LOC, N_BLK), jnp.float32),
        in_specs=[
            pl.BlockSpec(memory_space=pltpu.VMEM),
            pl.BlockSpec(memory_space=pltpu.HBM),
        ],
        out_specs=pl.BlockSpec(memory_space=pltpu.HBM),
        scratch_shapes=[
            pltpu.VMEM((2, K, W_TILE), jnp.float32),
            pltpu.VMEM((2, M_LOC, N_BLK), jnp.float32),
            pltpu.VMEM((3, M_LOC, N_BLK), jnp.float8_e4m3fn),
            pltpu.VMEM((N_DEV, M_LOC, N_BLK), jnp.float8_e4m3fn),
            pltpu.VMEM((8, 128), jnp.float32),
            pltpu.VMEM((N_DEV, 8, 128), jnp.float32),
            pltpu.HBM((N_DEV, M_LOC, N_BLK), jnp.float32),
            pltpu.SemaphoreType.DMA((2,)),
            pltpu.SemaphoreType.DMA((2,)),
            pltpu.SemaphoreType.DMA((2,)),
            pltpu.SemaphoreType.DMA((2,)),
            pltpu.SemaphoreType.DMA((4,)),
            pltpu.SemaphoreType.DMA((4,)),
            pltpu.SemaphoreType.DMA((4,)),
            pltpu.SemaphoreType.DMA((4,)),
        ],
        compiler_params=pltpu.CompilerParams(collective_id=0),
    )(x, w_mat)


# baseline (device time: 184170 ns/iter reference)
import jax
import jax.numpy as jnp
from jax import lax
from jax.experimental import pallas as pl
from jax.experimental.pallas import tpu as pltpu

N_DEV = 4
M_LOC = 1024
K = 4096
N_TOT = 8192
N_BLK = N_TOT // N_DEV
W_TILE = 256
N_TILES = N_TOT // W_TILE
TILES_PER_BLK = N_BLK // W_TILE

_PRECISION = lax.Precision.DEFAULT


def kernel(x, w_mat):
    def body(x_ref, w_ref, out_ref, y_hbm,
             w_buf, y_blk, q_send, q_recv, amax_mine, amax_all,
             w_sems, y_sems, r_sems, o_sems,
             ax_send_sems, ax_recv_sems, d_send_sems, d_recv_sems):
        my = lax.axis_index("i")

        bsem = pltpu.get_barrier_semaphore()
        for h in (1, 2, 3):
            pl.semaphore_signal(
                bsem, inc=1,
                device_id=((my + h) % N_DEV,),
                device_id_type=pl.DeviceIdType.MESH,
            )
        pl.semaphore_wait(bsem, 3)

        amax_all[...] = jnp.zeros((N_DEV, 8, 128), jnp.float32)

        def w_copy(t, slot):
            return pltpu.make_async_copy(
                w_ref.at[:, pl.ds(t * W_TILE, W_TILE)],
                w_buf.at[slot],
                w_sems.at[slot],
            )

        def y_store(j):
            return pltpu.make_async_copy(
                y_blk.at[j % 2], y_hbm.at[j], y_sems.at[j % 2])

        def w_copy_dyn(g, slot):
            return pltpu.make_async_copy(
                w_ref.at[:, pl.ds(g * W_TILE, W_TILE)],
                w_buf.at[slot],
                w_sems.at[slot],
            )

        w_copy(0, 0).start()
        amax = jnp.float32(0.0)
        for j in range(N_DEV):
            if j >= 2:
                y_store(j - 2).wait()

            def blk_step(t, amax_c, j=j):
                g = j * TILES_PER_BLK + t
                slot = lax.rem(g, 2)

                @pl.when(g + 1 < N_TILES)
                def _():
                    w_copy_dyn(g + 1, lax.rem(g + 1, 2)).start()

                w_copy_dyn(g, slot).wait()
                part = lax.dot_general(
                    x_ref[...], w_buf[slot],
                    dimension_numbers=(((1,), (0,)), ((), ())),
                    precision=_PRECISION,
                    preferred_element_type=jnp.float32,
                )
                y_blk[j % 2, :, pl.ds(t * W_TILE, W_TILE)] = part
                return jnp.maximum(amax_c, jnp.max(jnp.abs(part)))

            amax = lax.fori_loop(0, TILES_PER_BLK, blk_step, amax)
            y_store(j).start()

        amax_mine[...] = jnp.full((8, 128), amax, jnp.float32)
        for h in (1, 2, 3):
            pltpu.make_async_remote_copy(
                src_ref=amax_mine,
                dst_ref=amax_all.at[N_DEV - h],
                send_sem=ax_send_sems.at[h],
                recv_sem=ax_recv_sems.at[N_DEV - h],
                device_id=((my + h) % N_DEV,),
                device_id_type=pl.DeviceIdType.MESH,
            ).start()
        for r in (1, 2, 3):
            pltpu.make_async_remote_copy(
                src_ref=amax_mine,
                dst_ref=amax_all.at[r],
                send_sem=ax_send_sems.at[0],
                recv_sem=ax_recv_sems.at[r],
                device_id=(my,),
                device_id_type=pl.DeviceIdType.MESH,
            ).wait_recv()
        amax_g = jnp.maximum(amax, jnp.max(amax_all[...]))
        inv = 448.0 / amax_g
        scale = amax_g / 448.0

        def quant(v):
            return jnp.clip(v * inv, -448.0, 448.0).astype(jnp.float8_e4m3fn)

        for j in (2, 3):
            y_store(j).wait()

        def y_read(jb, slot):
            return pltpu.make_async_copy(
                y_hbm.at[jb], y_blk.at[slot], r_sems.at[slot])

        y_read((my + 1) % N_DEV, 0).start()
        y_read((my + 2) % N_DEV, 1).start()
        for h in (1, 2, 3):
            slot = (h - 1) % 2
            y_read((my + h) % N_DEV, slot).wait()
            q_send[h - 1, :, :] = quant(y_blk[slot])
            pltpu.make_async_remote_copy(
                src_ref=q_send.at[h - 1],
                dst_ref=q_recv.at[N_DEV - h],
                send_sem=d_send_sems.at[h],
                recv_sem=d_recv_sems.at[N_DEV - h],
                device_id=((my + h) % N_DEV,),
                device_id_type=pl.DeviceIdType.MESH,
            ).start()
            if h == 1:
                y_read((my + 3) % N_DEV, 0).start()
            elif h == 2:
                y_read(my, 1).start()

        def out_store(s_idx, slot):
            return pltpu.make_async_copy(
                y_blk.at[slot],
                out_ref.at[pl.ds(s_idx * M_LOC, M_LOC), :],
                o_sems.at[slot],
            )

        y_read(my, 1).wait()
        y_blk[1, :, :] = quant(y_blk[1]).astype(jnp.float32) * scale
        own_store = out_store(my, 1)
        own_store.start()

        prev = {0: None, 1: own_store}
        for n, r in enumerate((3, 2, 1)):
            slot = n % 2
            pltpu.make_async_remote_copy(
                src_ref=q_send.at[0],
                dst_ref=q_recv.at[r],
                send_sem=d_send_sems.at[0],
                recv_sem=d_recv_sems.at[r],
                device_id=(my,),
                device_id_type=pl.DeviceIdType.MESH,
            ).wait_recv()
            if prev[slot] is not None:
                prev[slot].wait()
            y_blk[slot, :, :] = q_recv[r].astype(jnp.float32) * scale
            st = out_store((my + r) % N_DEV, slot)
            st.start()
            prev[slot] = st
        prev[0].wait()
        prev[1].wait()

        for h in (1, 2, 3):
            pltpu.make_async_remote_copy(
                src_ref=amax_mine,
                dst_ref=amax_all.at[N_DEV - h],
                send_sem=ax_send_sems.at[h],
                recv_sem=ax_recv_sems.at[0],
                device_id=((my + h) % N_DEV,),
                device_id_type=pl.DeviceIdType.MESH,
            ).wait_send()
            pltpu.make_async_remote_copy(
                src_ref=q_send.at[h - 1],
                dst_ref=q_recv.at[N_DEV - h],
                send_sem=d_send_sems.at[h],
                recv_sem=d_recv_sems.at[0],
                device_id=((my + h) % N_DEV,),
                device_id_type=pl.DeviceIdType.MESH,
            ).wait_send()

    out, _ = pl.pallas_call(
        body,
        out_shape=(
            jax.ShapeDtypeStruct((N_DEV * M_LOC, N_BLK), jnp.float32),
            jax.ShapeDtypeStruct((N_DEV, M_LOC, N_BLK), jnp.float32),
        ),
        in_specs=[
            pl.BlockSpec(memory_space=pltpu.VMEM),
            pl.BlockSpec(memory_space=pltpu.HBM),
        ],
        out_specs=(
            pl.BlockSpec(memory_space=pltpu.HBM),
            pl.BlockSpec(memory_space=pltpu.HBM),
        ),
        scratch_shapes=[
            pltpu.VMEM((2, K, W_TILE), jnp.float32),
            pltpu.VMEM((2, M_LOC, N_BLK), jnp.float32),
            pltpu.VMEM((3, M_LOC, N_BLK), jnp.float8_e4m3fn),
            pltpu.VMEM((N_DEV, M_LOC, N_BLK), jnp.float8_e4m3fn),
            pltpu.VMEM((8, 128), jnp.float32),
            pltpu.VMEM((N_DEV, 8, 128), jnp.float32),
            pltpu.SemaphoreType.DMA((2,)),
            pltpu.SemaphoreType.DMA((2,)),
            pltpu.SemaphoreType.DMA((2,)),
            pltpu.SemaphoreType.DMA((2,)),
            pltpu.SemaphoreType.DMA((4,)),
            pltpu.SemaphoreType.DMA((4,)),
            pltpu.SemaphoreType.DMA((4,)),
            pltpu.SemaphoreType.DMA((4,)),
        ],
        compiler_params=pltpu.CompilerParams(
            collective_id=0,
            vmem_limit_bytes=100 * 1024 * 1024,
        ),
    )(x, w_mat)
    return out
